# baseline (speedup 1.0000x reference)
import functools

import jax
import jax.numpy as jnp
from jax import lax
from jax.experimental import pallas as pl
from jax.experimental.pallas import tpu as pltpu
from jax.experimental.pallas import tpu_sc as plsc

MID = 128
B = 16384
L = 20
NW = 32
BPW = B // NW
GRP = BPW // 16

V1 = 8
V2 = 3000
V3 = 1000


LPAD = 24


def _tc_tables(emb1_ref, emb2_ref, emb3_ref, w1_ref, w2_ref,
               acT_ref, pcT_ref, atT_ref, dfT_ref, evT_ref,
               t_ref, ac_ref, pc_ref, at_ref, df_ref, ev_ref):
    ac_ref[pl.ds(0, L), :] = acT_ref[...]
    pc_ref[pl.ds(0, L), :] = pcT_ref[...]
    at_ref[pl.ds(0, L), :] = atT_ref[...]
    df_ref[pl.ds(0, L), :] = dfT_ref[...]
    ev_ref[pl.ds(0, L), :] = evT_ref[...]

    u = w2_ref[...] @ w1_ref[...]

    def proj(x, lo):
        return lax.dot_general(u[:, lo:lo + 128], x,
                               (((1,), (1,)), ((), ())))[0]

    t_ref[pl.ds(0, 5)] = proj(emb1_ref[...], 128)
    t_ref[pl.ds(8, V2)] = proj(emb2_ref[...], 256)
    t_ref[pl.ds(8 + V2, V3)] = proj(emb3_ref[...], 384)
    t_ref[pl.ds(8 + V2 + V3, V3)] = proj(emb3_ref[...], 512)
    t_ref[pl.ds(8 + V2 + 2 * V3, V3)] = proj(emb3_ref[...], 640)


def _tc_states(states_ref, w1_ref, b1_ref, w2_ref, b2_ref, s_ref):
    u = w2_ref[...] @ w1_ref[...]
    c = jnp.sum(w2_ref[...] * b1_ref[...][None, :]) + b2_ref[0]
    s_ref[...] = jnp.sum(states_ref[...] * u[:, 0:128], axis=1) + c


def _sc_gather(ac_hbm, pc_hbm, at_hbm, df_hbm, ev_hbm, t_hbm,
               out_hbm,
               ac_v, pc_v, at_v, df_v, ev_v,
               t1_v, t2_v, t3a_v, t3b_v, t3c_v, o_v,
               sem0, sem1, sem2, sem3, sem4):
    wid = lax.axis_index("s") * 2 + lax.axis_index("c")
    base = wid * BPW
    plan = ((ac_hbm, ac_v, t1_v, sem0), (pc_hbm, pc_v, t2_v, sem1),
            (at_hbm, at_v, t3a_v, sem2), (df_hbm, df_v, t3b_v, sem3),
            (ev_hbm, ev_v, t3c_v, sem4))
    copies = []
    for hbm, v, _, sem in plan:
        cps = [pltpu.async_copy(hbm.at[pl.ds(j * B + base, BPW)],
                                v.at[pl.ds(j * BPW, BPW)], sem)
               for j in range(L)]
        copies.append(cps)
    pltpu.sync_copy(t_hbm.at[pl.ds(0, V1)], t1_v)
    pltpu.sync_copy(t_hbm.at[pl.ds(8, V2)], t2_v)
    pltpu.sync_copy(t_hbm.at[pl.ds(8 + V2, V3)], t3a_v)
    pltpu.sync_copy(t_hbm.at[pl.ds(8 + V2 + V3, V3)], t3b_v)
    pltpu.sync_copy(t_hbm.at[pl.ds(8 + V2 + 2 * V3, V3)], t3c_v)

    for cps in copies:
        for cp in cps:
            cp.wait()

    def body(g, carry):
        acc = jnp.zeros((16,), jnp.float32)
        for j in range(L):
            off = j * BPW + g * 16
            for _, iv, tv, _ in plan:
                ids = iv[pl.ds(off, 16)]
                acc = acc + plsc.load_gather(tv, [ids])
        o_v[pl.ds(g * 16, 16)] = acc
        return carry

    lax.fori_loop(0, GRP, body, 0)
    pltpu.sync_copy(o_v, out_hbm.at[pl.ds(base, BPW)])


_sc_call = functools.partial(
    pl.kernel,
    out_type=jax.ShapeDtypeStruct((B,), jnp.float32),
    mesh=plsc.VectorSubcoreMesh(core_axis_name="c", subcore_axis_name="s"),
    scratch_types=[
        pltpu.VMEM((BPW * L,), jnp.int32),
        pltpu.VMEM((BPW * L,), jnp.int32),
        pltpu.VMEM((BPW * L,), jnp.int32),
        pltpu.VMEM((BPW * L,), jnp.int32),
        pltpu.VMEM((BPW * L,), jnp.int32),
        pltpu.VMEM((V1,), jnp.float32),
        pltpu.VMEM((V2,), jnp.float32),
        pltpu.VMEM((V3,), jnp.float32),
        pltpu.VMEM((V3,), jnp.float32),
        pltpu.VMEM((V3,), jnp.float32),
        pltpu.VMEM((BPW,), jnp.float32),
        pltpu.SemaphoreType.DMA,
        pltpu.SemaphoreType.DMA,
        pltpu.SemaphoreType.DMA,
        pltpu.SemaphoreType.DMA,
        pltpu.SemaphoreType.DMA,
    ],
    compiler_params=pltpu.CompilerParams(needs_layout_passes=False),
)(_sc_gather)


def kernel(states, action_categories, play_card_ids, attacking_card_ids,
           attacked_card_ids, evolving_card_ids, emb1, emb2, emb3, W1, b1,
           W2, b2):
    acT = jnp.asarray(action_categories, jnp.int32).T
    pcT = jnp.asarray(play_card_ids, jnp.int32).T
    atT = jnp.asarray(attacking_card_ids, jnp.int32).T
    dfT = jnp.asarray(attacked_card_ids, jnp.int32).T
    evT = jnp.asarray(evolving_card_ids, jnp.int32).T

    idx_pad = jax.ShapeDtypeStruct((LPAD, B), jnp.int32)
    t_all, ac2, pc2, at2, df2, ev2 = pl.pallas_call(
        _tc_tables,
        out_shape=(jax.ShapeDtypeStruct((6016,), jnp.float32),
                   idx_pad, idx_pad, idx_pad, idx_pad, idx_pad),
    )(emb1, emb2, emb3, W1, W2, acT, pcT, atT, dfT, evT)
    ac = ac2.reshape(LPAD * B)
    pc = pc2.reshape(LPAD * B)
    at = at2.reshape(LPAD * B)
    df = df2.reshape(LPAD * B)
    ev = ev2.reshape(LPAD * B)

    s = pl.pallas_call(
        _tc_states,
        out_shape=jax.ShapeDtypeStruct((B,), jnp.float32),
    )(states, W1, b1, W2, b2)

    partial = _sc_call(ac, pc, at, df, ev, t_all)
    return (partial + s).reshape(B, 1)


# speedup vs baseline: 1.1621x; 1.1621x over previous
import functools

import jax
import jax.numpy as jnp
from jax import lax
from jax.experimental import pallas as pl
from jax.experimental.pallas import tpu as pltpu
from jax.experimental.pallas import tpu_sc as plsc

MID = 128
B = 16384
L = 20
NW = 32
BPW = B // NW
GRP = BPW // 16

V1 = 8
V2 = 3000
V3 = 1000


def _tc_tables(emb1_ref, emb2_ref, emb3_ref, w1_ref, w2_ref, t_ref):
    u = w2_ref[...] @ w1_ref[...]

    def proj(x, lo):
        return lax.dot_general(u[:, lo:lo + 128], x,
                               (((1,), (1,)), ((), ())))[0]

    t_ref[pl.ds(0, 5)] = proj(emb1_ref[...], 128)
    t_ref[pl.ds(8, V2)] = proj(emb2_ref[...], 256)
    t_ref[pl.ds(8 + V2, V3)] = proj(emb3_ref[...], 384)
    t_ref[pl.ds(8 + V2 + V3, V3)] = proj(emb3_ref[...], 512)
    t_ref[pl.ds(8 + V2 + 2 * V3, V3)] = proj(emb3_ref[...], 640)


def _tc_states(states_ref, w1_ref, b1_ref, w2_ref, b2_ref, s_ref):
    u = w2_ref[...] @ w1_ref[...]
    c = jnp.sum(w2_ref[...] * b1_ref[...][None, :]) + b2_ref[0]
    s_ref[...] = jnp.sum(states_ref[...] * u[:, 0:128], axis=1) + c


def _sc_gather(ac_hbm, pc_hbm, at_hbm, df_hbm, ev_hbm, t_hbm,
               out_hbm,
               ac_v, pc_v, at_v, df_v, ev_v,
               t1_v, t2_v, t3a_v, t3b_v, t3c_v, o_v,
               sem0, sem1, sem2, sem3, sem4):
    wid = lax.axis_index("s") * 2 + lax.axis_index("c")
    base = wid * BPW
    plan = ((ac_hbm, ac_v, t1_v, sem0), (pc_hbm, pc_v, t2_v, sem1),
            (at_hbm, at_v, t3a_v, sem2), (df_hbm, df_v, t3b_v, sem3),
            (ev_hbm, ev_v, t3c_v, sem4))
    copies = []
    for hbm, v, _, sem in plan:
        cps = [pltpu.async_copy(hbm.at[pl.ds(j * B + base, BPW)],
                                v.at[pl.ds(j * BPW, BPW)], sem)
               for j in range(L)]
        copies.append(cps)
    tcopies = [
        pltpu.async_copy(t_hbm.at[pl.ds(0, V1)], t1_v, sem0),
        pltpu.async_copy(t_hbm.at[pl.ds(8, V2)], t2_v, sem1),
        pltpu.async_copy(t_hbm.at[pl.ds(8 + V2, V3)], t3a_v, sem2),
        pltpu.async_copy(t_hbm.at[pl.ds(8 + V2 + V3, V3)], t3b_v, sem3),
        pltpu.async_copy(t_hbm.at[pl.ds(8 + V2 + 2 * V3, V3)], t3c_v, sem4),
    ]
    for cps in copies:
        for cp in cps:
            cp.wait()
    for cp in tcopies:
        cp.wait()

    def body(g, carry):
        acc = jnp.zeros((16,), jnp.float32)
        for j in range(L):
            off = j * BPW + g * 16
            for _, iv, tv, _ in plan:
                ids = iv[pl.ds(off, 16)]
                acc = acc + plsc.load_gather(tv, [ids])
        o_v[pl.ds(g * 16, 16)] = acc
        return carry

    lax.fori_loop(0, GRP, body, 0)
    pltpu.sync_copy(o_v, out_hbm.at[pl.ds(base, BPW)])


_sc_call = functools.partial(
    pl.kernel,
    out_type=jax.ShapeDtypeStruct((B,), jnp.float32),
    mesh=plsc.VectorSubcoreMesh(core_axis_name="c", subcore_axis_name="s"),
    scratch_types=[
        pltpu.VMEM((BPW * L,), jnp.int32),
        pltpu.VMEM((BPW * L,), jnp.int32),
        pltpu.VMEM((BPW * L,), jnp.int32),
        pltpu.VMEM((BPW * L,), jnp.int32),
        pltpu.VMEM((BPW * L,), jnp.int32),
        pltpu.VMEM((V1,), jnp.float32),
        pltpu.VMEM((V2,), jnp.float32),
        pltpu.VMEM((V3,), jnp.float32),
        pltpu.VMEM((V3,), jnp.float32),
        pltpu.VMEM((V3,), jnp.float32),
        pltpu.VMEM((BPW,), jnp.float32),
        pltpu.SemaphoreType.DMA,
        pltpu.SemaphoreType.DMA,
        pltpu.SemaphoreType.DMA,
        pltpu.SemaphoreType.DMA,
        pltpu.SemaphoreType.DMA,
    ],
    compiler_params=pltpu.CompilerParams(needs_layout_passes=False),
)(_sc_gather)


def kernel(states, action_categories, play_card_ids, attacking_card_ids,
           attacked_card_ids, evolving_card_ids, emb1, emb2, emb3, W1, b1,
           W2, b2):
    ac = jnp.asarray(action_categories, jnp.int32).T.reshape(B * L)
    pc = jnp.asarray(play_card_ids, jnp.int32).T.reshape(B * L)
    at = jnp.asarray(attacking_card_ids, jnp.int32).T.reshape(B * L)
    df = jnp.asarray(attacked_card_ids, jnp.int32).T.reshape(B * L)
    ev = jnp.asarray(evolving_card_ids, jnp.int32).T.reshape(B * L)

    t_all = pl.pallas_call(
        _tc_tables,
        out_shape=jax.ShapeDtypeStruct((6016,), jnp.float32),
    )(emb1, emb2, emb3, W1, W2)

    s = pl.pallas_call(
        _tc_states,
        out_shape=jax.ShapeDtypeStruct((B,), jnp.float32),
    )(states, W1, b1, W2, b2)

    partial = _sc_call(ac, pc, at, df, ev, t_all)
    return (partial + s).reshape(B, 1)
